# Initial kernel scaffold; baseline (speedup 1.0000x reference)
import jax
import jax.numpy as jnp
from jax.experimental import pallas as pl


def kernel(input, adj, M, W, b, a):
    raise NotImplementedError("write your pallas kernel here")


# speedup vs baseline: 3.1157x; 3.1157x over previous
import jax
import jax.numpy as jnp
from jax.experimental import pallas as pl

N = 10000
D = 128
TM = 400


def _node_kernel(x_ref, w_ref, b_ref, a_ref, xp_ref, v_ref):
    xp = jnp.dot(x_ref[...], w_ref[...], preferred_element_type=jnp.float32,
                 precision=jax.lax.Precision.HIGHEST)
    h = jnp.tanh(xp + b_ref[...])
    e = jnp.dot(h, a_ref[...], preferred_element_type=jnp.float32,
                precision=jax.lax.Precision.HIGHEST)
    valid = jnp.sum(h, axis=1, keepdims=True) != 0.0
    c = jnp.sum(jnp.abs(a_ref[...]))
    v = jnp.where(valid, jnp.exp(e - c), 0.0)
    xp_ref[...] = xp
    v_ref[...] = v


def _attn_kernel(adj_ref, xp_ref, vr_ref, out_ref):
    aw = jnp.where(adj_ref[...] > 0.0, vr_ref[...], 0.0)
    num = jax.lax.dot_general(
        aw, xp_ref[...], (((1,), (0,)), ((), ())),
        preferred_element_type=jnp.float32)
    den = jnp.sum(aw, axis=1, keepdims=True)
    out_ref[...] = jnp.where(den > 0.0, num / den, 0.0)


def kernel(input, adj, M, W, b, a):
    x = input.astype(jnp.float32)
    b2 = b.reshape(1, D).astype(jnp.float32)
    xp, v = pl.pallas_call(
        _node_kernel,
        out_shape=(
            jax.ShapeDtypeStruct((N, D), jnp.float32),
            jax.ShapeDtypeStruct((N, 1), jnp.float32),
        ),
    )(x, W.astype(jnp.float32), b2, a.astype(jnp.float32))
    vr = v.reshape(1, N)
    out = pl.pallas_call(
        _attn_kernel,
        grid=(N // TM,),
        in_specs=[
            pl.BlockSpec((TM, N), lambda i: (i, 0)),
            pl.BlockSpec((N, D), lambda i: (0, 0)),
            pl.BlockSpec((1, N), lambda i: (0, 0)),
        ],
        out_specs=pl.BlockSpec((TM, D), lambda i: (i, 0)),
        out_shape=jax.ShapeDtypeStruct((N, D), jnp.float32),
    )(adj, xp, vr)
    row_in_range = jnp.arange(N, dtype=jnp.int32)[:, None] < M
    return jnp.where(row_in_range, out, 0.0)


# speedup vs baseline: 3.3566x; 1.0773x over previous
import jax
import jax.numpy as jnp
from jax.experimental import pallas as pl

N = 10000
D = 128
TM = 400


def _node_kernel(x_ref, w_ref, b_ref, a_ref, xp_ref, v_ref):
    xp = jnp.dot(x_ref[...], w_ref[...], preferred_element_type=jnp.float32)
    h = jnp.tanh(xp + b_ref[...])
    e = jnp.dot(h, a_ref[...], preferred_element_type=jnp.float32)
    valid = jnp.sum(h, axis=1, keepdims=True) != 0.0
    c = jnp.sum(jnp.abs(a_ref[...]))
    v = jnp.where(valid, jnp.exp(e - c), 0.0)
    xp_ref[...] = xp
    v_ref[...] = v


def _attn_kernel(adj_ref, xp_ref, vr_ref, out_ref):
    aw = jnp.where(adj_ref[...] > 0.0, vr_ref[...], 0.0)
    num = jax.lax.dot_general(
        aw, xp_ref[...], (((1,), (0,)), ((), ())),
        preferred_element_type=jnp.float32)
    den = jnp.sum(aw, axis=1, keepdims=True)
    out_ref[...] = jnp.where(den > 0.0, num / den, 0.0)


def kernel(input, adj, M, W, b, a):
    x = input.astype(jnp.float32)
    b2 = b.reshape(1, D).astype(jnp.float32)
    xp, v = pl.pallas_call(
        _node_kernel,
        out_shape=(
            jax.ShapeDtypeStruct((N, D), jnp.float32),
            jax.ShapeDtypeStruct((N, 1), jnp.float32),
        ),
    )(x, W.astype(jnp.float32), b2, a.astype(jnp.float32))
    vr = v.reshape(1, N)
    out = pl.pallas_call(
        _attn_kernel,
        grid=(N // TM,),
        in_specs=[
            pl.BlockSpec((TM, N), lambda i: (i, 0)),
            pl.BlockSpec((N, D), lambda i: (0, 0)),
            pl.BlockSpec((1, N), lambda i: (0, 0)),
        ],
        out_specs=pl.BlockSpec((TM, D), lambda i: (i, 0)),
        out_shape=jax.ShapeDtypeStruct((N, D), jnp.float32),
    )(adj, xp, vr)
    row_in_range = jnp.arange(N, dtype=jnp.int32)[:, None] < M
    return jnp.where(row_in_range, out, 0.0)


# speedup vs baseline: 3.5729x; 1.0644x over previous
import jax
import jax.numpy as jnp
from jax.experimental import pallas as pl
from jax.experimental.pallas import tpu as pltpu

N = 10000
D = 128
TM = 400


def _node_kernel(x_ref, w_ref, b_ref, a_ref, xp_ref, v_ref):
    xp = jnp.dot(x_ref[...], w_ref[...], preferred_element_type=jnp.float32)
    h = jnp.tanh(xp + b_ref[...])
    e = jnp.dot(h, a_ref[...], preferred_element_type=jnp.float32)
    valid = jnp.sum(h, axis=1, keepdims=True) != 0.0
    c = jnp.sum(jnp.abs(a_ref[...]))
    v = jnp.where(valid, jnp.exp(e - c), 0.0)
    xp_ref[...] = xp
    v_ref[...] = v


def _attn_kernel(m_ref, adj_ref, xp_ref, vr_ref, out_ref):
    aw = jnp.where(adj_ref[...] > 0.0, vr_ref[...], 0.0)
    num = jax.lax.dot_general(
        aw, xp_ref[...], (((1,), (0,)), ((), ())),
        preferred_element_type=jnp.float32)
    den = jnp.sum(aw, axis=1, keepdims=True)
    row = (pl.program_id(0) * TM
           + jax.lax.broadcasted_iota(jnp.int32, (TM, 1), 0))
    keep = (den > 0.0) & (row < m_ref[0])
    out_ref[...] = jnp.where(keep, num / den, 0.0)


def kernel(input, adj, M, W, b, a):
    x = input.astype(jnp.float32)
    b2 = b.reshape(1, D).astype(jnp.float32)
    xp, v = pl.pallas_call(
        _node_kernel,
        out_shape=(
            jax.ShapeDtypeStruct((N, D), jnp.float32),
            jax.ShapeDtypeStruct((N, 1), jnp.float32),
        ),
    )(x, W.astype(jnp.float32), b2, a.astype(jnp.float32))
    vr = v.reshape(1, N)
    m_arr = jnp.asarray(M, dtype=jnp.int32).reshape(1)
    out = pl.pallas_call(
        _attn_kernel,
        grid=(N // TM,),
        in_specs=[
            pl.BlockSpec(memory_space=pltpu.SMEM),
            pl.BlockSpec((TM, N), lambda i: (i, 0)),
            pl.BlockSpec((N, D), lambda i: (0, 0)),
            pl.BlockSpec((1, N), lambda i: (0, 0)),
        ],
        out_specs=pl.BlockSpec((TM, D), lambda i: (i, 0)),
        out_shape=jax.ShapeDtypeStruct((N, D), jnp.float32),
    )(m_arr, adj, xp, vr)
    return out
